# Initial kernel scaffold; baseline (speedup 1.0000x reference)
import jax
import jax.numpy as jnp
from jax.experimental import pallas as pl


def kernel(x, edge_index, batch, W1, b1, W2, b2):
    raise NotImplementedError("write your pallas kernel here")


# speedup vs baseline: 12.4217x; 12.4217x over previous
import functools

import jax
import jax.numpy as jnp
from jax import lax
from jax.experimental import pallas as pl
from jax.experimental.pallas import tpu as pltpu
from jax.experimental.pallas import tpu_sc as plsc

_N = 10000
_E = 320000
_D = 128
_G = 16
_NC = 2
_NS = 16
_NW = _NC * _NS
_EPW = _E // _NW
_CH = 80
_NCHUNK = _EPW // _CH
_STRIPE = 624
_LAST = _N - (_NS - 1) * _STRIPE

_ROWS = 1000
_NBLK = _N // _ROWS

_HIGH = jax.lax.Precision.HIGHEST


def _striped(sid, copy):
    @pl.when(sid < _NS - 1)
    def _():
        copy(sid * _STRIPE, _STRIPE)

    @pl.when(sid == _NS - 1)
    def _():
        copy((_NS - 1) * _STRIPE, _LAST)


def _sc_mesh():
    return plsc.VectorSubcoreMesh(
        core_axis_name="c", subcore_axis_name="s",
        num_cores=_NC, num_subcores=_NS)


def _hist_body(dst_hbm, zeros_hbm, ones_hbm, out_hbm, idx_v, ones_v, acc_sh):
    cid = lax.axis_index("c")
    sid = lax.axis_index("s")
    wid = cid * _NS + sid
    pltpu.sync_copy(ones_hbm, ones_v)
    _striped(sid, lambda lo, n: pltpu.sync_copy(
        zeros_hbm.at[pl.ds(lo, n)], acc_sh.at[pl.ds(lo, n)]))
    plsc.subcore_barrier()
    base = wid * _EPW

    @pl.loop(0, _NCHUNK)
    def _(g):
        pltpu.sync_copy(dst_hbm.at[pl.ds(base + g * _CH, _CH)], idx_v)
        pltpu.sync_copy(ones_v, acc_sh.at[idx_v], add=True)

    plsc.subcore_barrier()
    _striped(sid, lambda lo, n: pltpu.sync_copy(
        acc_sh.at[pl.ds(lo, n)], out_hbm.at[cid, pl.ds(lo, n)]))


def _sc_degree(dst, zeros_nd, ones_cd):
    fn = pl.kernel(
        _hist_body,
        out_type=jax.ShapeDtypeStruct((_NC, _N, _D), jnp.float32),
        mesh=_sc_mesh(),
        scratch_types=[
            pltpu.VMEM((_CH,), jnp.int32),
            pltpu.VMEM((_CH, _D), jnp.float32),
            pltpu.VMEM_SHARED((_N, _D), jnp.float32),
        ])
    return fn(dst, zeros_nd, ones_cd)


def _edge_body(hs_hbm, src_hbm, dst_hbm, out_hbm, sidx_v, didx_v, rows_v,
               acc_sh, sem):
    cid = lax.axis_index("c")
    sid = lax.axis_index("s")
    wid = cid * _NS + sid
    _striped(sid, lambda lo, n: pltpu.sync_copy(
        hs_hbm.at[pl.ds(lo, n)], acc_sh.at[pl.ds(lo, n)]))
    plsc.subcore_barrier()
    base = wid * _EPW

    @pl.loop(0, _NCHUNK)
    def _(g):
        off = base + g * _CH
        pltpu.sync_copy(src_hbm.at[pl.ds(off, _CH)], sidx_v)
        pltpu.sync_copy(dst_hbm.at[pl.ds(off, _CH)], didx_v)
        pltpu.async_copy(hs_hbm.at[sidx_v], rows_v, sem).wait()
        pltpu.sync_copy(rows_v, acc_sh.at[didx_v], add=True)

    plsc.subcore_barrier()
    _striped(sid, lambda lo, n: pltpu.sync_copy(
        acc_sh.at[pl.ds(lo, n)], out_hbm.at[cid, pl.ds(lo, n)]))


def _sc_edge(hs, src, dst):
    fn = pl.kernel(
        _edge_body,
        out_type=jax.ShapeDtypeStruct((_NC, _N, _D), jnp.float32),
        mesh=_sc_mesh(),
        scratch_types=[
            pltpu.VMEM((_CH,), jnp.int32),
            pltpu.VMEM((_CH,), jnp.int32),
            pltpu.VMEM((_CH, _D), jnp.float32),
            pltpu.VMEM_SHARED((_N, _D), jnp.float32),
            pltpu.SemaphoreType.DMA,
        ])
    return fn(hs, src, dst)


def _dinv_block(degp_ref):
    deg = degp_ref[0, :, 0:1] + degp_ref[1, :, 0:1] + 1.0
    return 1.0 / jnp.sqrt(jnp.maximum(deg, 1.0))


def _k2_body(x_ref, w_ref, degp_ref, o_ref):
    dinv = _dinv_block(degp_ref)
    h = jnp.dot(x_ref[...], w_ref[...],
                preferred_element_type=jnp.float32, precision=_HIGH)
    o_ref[...] = h * dinv


def _tc_scale1(x, W1, degp):
    return pl.pallas_call(
        _k2_body,
        grid=(_NBLK,),
        in_specs=[
            pl.BlockSpec((_ROWS, _D), lambda i: (i, 0)),
            pl.BlockSpec((_D, _D), lambda i: (0, 0)),
            pl.BlockSpec((_NC, _ROWS, _D), lambda i: (0, i, 0)),
        ],
        out_specs=pl.BlockSpec((_ROWS, _D), lambda i: (i, 0)),
        out_shape=jax.ShapeDtypeStruct((_N, _D), jnp.float32),
    )(x, W1, degp)


def _k3_body(acc_ref, hs1_ref, degp_ref, w_ref, b_ref, o_ref):
    dinv = _dinv_block(degp_ref)
    agg = acc_ref[0] + acc_ref[1] - hs1_ref[...]
    out1 = jnp.maximum(agg * dinv + b_ref[...], 0.0)
    h2 = jnp.dot(out1, w_ref[...],
                 preferred_element_type=jnp.float32, precision=_HIGH)
    o_ref[...] = h2 * dinv


def _tc_mid(acc1, hs1, degp, W2, b1r):
    return pl.pallas_call(
        _k3_body,
        grid=(_NBLK,),
        in_specs=[
            pl.BlockSpec((_NC, _ROWS, _D), lambda i: (0, i, 0)),
            pl.BlockSpec((_ROWS, _D), lambda i: (i, 0)),
            pl.BlockSpec((_NC, _ROWS, _D), lambda i: (0, i, 0)),
            pl.BlockSpec((_D, _D), lambda i: (0, 0)),
            pl.BlockSpec((1, _D), lambda i: (0, 0)),
        ],
        out_specs=pl.BlockSpec((_ROWS, _D), lambda i: (i, 0)),
        out_shape=jax.ShapeDtypeStruct((_N, _D), jnp.float32),
    )(acc1, hs1, degp, W2, b1r)


def _k4_body(acc_ref, hs2_ref, degp_ref, b_ref, batch_ref,
             node_ref, gout_ref, seg_ref, cnt_ref):
    i = pl.program_id(0)
    dinv = _dinv_block(degp_ref)
    node = (acc_ref[0] + acc_ref[1] - hs2_ref[...]) * dinv + b_ref[...]
    node_ref[...] = node
    gids = lax.broadcasted_iota(jnp.int32, (_G, _ROWS), 0)
    oh = (gids == batch_ref[0]).astype(jnp.float32)
    seg_p = jnp.dot(oh, node,
                    preferred_element_type=jnp.float32, precision=_HIGH)
    cnt_p = jnp.sum(oh, axis=1, keepdims=True)

    @pl.when(i == 0)
    def _():
        seg_ref[...] = seg_p
        cnt_ref[...] = jnp.broadcast_to(cnt_p, (_G, _D))

    @pl.when(i > 0)
    def _():
        seg_ref[...] += seg_p
        cnt_ref[...] += jnp.broadcast_to(cnt_p, (_G, _D))

    @pl.when(i == _NBLK - 1)
    def _():
        gout_ref[...] = seg_ref[...] / jnp.maximum(cnt_ref[...], 1.0)


def _tc_final(acc2, hs2, degp, b2r, batch3):
    return pl.pallas_call(
        _k4_body,
        grid=(_NBLK,),
        in_specs=[
            pl.BlockSpec((_NC, _ROWS, _D), lambda i: (0, i, 0)),
            pl.BlockSpec((_ROWS, _D), lambda i: (i, 0)),
            pl.BlockSpec((_NC, _ROWS, _D), lambda i: (0, i, 0)),
            pl.BlockSpec((1, _D), lambda i: (0, 0)),
            pl.BlockSpec((1, 1, _ROWS), lambda i: (i, 0, 0)),
        ],
        out_specs=[
            pl.BlockSpec((_ROWS, _D), lambda i: (i, 0)),
            pl.BlockSpec((_G, _D), lambda i: (0, 0)),
        ],
        out_shape=[
            jax.ShapeDtypeStruct((_N, _D), jnp.float32),
            jax.ShapeDtypeStruct((_G, _D), jnp.float32),
        ],
        scratch_shapes=[
            pltpu.VMEM((_G, _D), jnp.float32),
            pltpu.VMEM((_G, _D), jnp.float32),
        ],
    )(acc2, hs2, degp, b2r, batch3)


def kernel(x, edge_index, batch, W1, b1, W2, b2):
    src = edge_index[0]
    dst = edge_index[1]
    zeros_nd = jnp.zeros((_N, _D), jnp.float32)
    ones_cd = jnp.ones((_CH, _D), jnp.float32)
    degp = _sc_degree(dst, zeros_nd, ones_cd)
    hs1 = _tc_scale1(x, W1, degp)
    acc1 = _sc_edge(hs1, src, dst)
    hs2 = _tc_mid(acc1, hs1, degp, W2, b1.reshape(1, _D))
    acc2 = _sc_edge(hs2, src, dst)
    node_emb, graph_emb = _tc_final(
        acc2, hs2, degp, b2.reshape(1, _D), batch.reshape(_NBLK, 1, _ROWS))
    return node_emb, graph_emb


# speedup vs baseline: 25.5616x; 2.0578x over previous
import functools

import jax
import jax.numpy as jnp
from jax import lax
from jax.experimental import pallas as pl
from jax.experimental.pallas import tpu as pltpu
from jax.experimental.pallas import tpu_sc as plsc

_N = 10000
_E = 320000
_D = 128
_G = 16
_NC = 2
_NS = 16
_NW = _NC * _NS
_EPW = _E // _NW
_CH = 80
_NCHUNK = _EPW // _CH
_STRIPE = 624
_LAST = _N - (_NS - 1) * _STRIPE

_ROWS = 1000
_NBLK = _N // _ROWS

_HIGH = jax.lax.Precision.HIGHEST


def _striped(sid, copy):
    @pl.when(sid < _NS - 1)
    def _():
        copy(sid * _STRIPE, _STRIPE)

    @pl.when(sid == _NS - 1)
    def _():
        copy((_NS - 1) * _STRIPE, _LAST)


def _sc_mesh():
    return plsc.VectorSubcoreMesh(
        core_axis_name="c", subcore_axis_name="s",
        num_cores=_NC, num_subcores=_NS)


def _hist_body(dst3_hbm, zeros_hbm, ones_hbm, out_hbm, didx_v, ones_v, acc_sh):
    cid = lax.axis_index("c")
    sid = lax.axis_index("s")
    wid = cid * _NS + sid
    pltpu.sync_copy(ones_hbm, ones_v)
    pltpu.sync_copy(dst3_hbm.at[wid], didx_v)
    _striped(sid, lambda lo, n: pltpu.sync_copy(
        zeros_hbm.at[pl.ds(lo, n)], acc_sh.at[pl.ds(lo, n)]))
    plsc.subcore_barrier()

    @pl.loop(0, _NCHUNK)
    def _(g):
        pltpu.sync_copy(ones_v, acc_sh.at[didx_v.at[g]], add=True)

    plsc.subcore_barrier()
    _striped(sid, lambda lo, n: pltpu.sync_copy(
        acc_sh.at[pl.ds(lo, n)], out_hbm.at[cid, pl.ds(lo, n)]))


def _sc_degree(dst3, zeros_nd, ones_cd):
    fn = pl.kernel(
        _hist_body,
        out_type=jax.ShapeDtypeStruct((_NC, _N, _D), jnp.float32),
        mesh=_sc_mesh(),
        scratch_types=[
            pltpu.VMEM((_NCHUNK, _CH), jnp.int32),
            pltpu.VMEM((_CH, _D), jnp.float32),
            pltpu.VMEM_SHARED((_N, _D), jnp.float32),
        ])
    return fn(dst3, zeros_nd, ones_cd)


def _edge_body(hs_hbm, src3_hbm, dst_hbm, out_hbm, sidx_v, d0_v, d1_v,
               rows0_v, rows1_v, acc_sh, sem_d0, sem_d1, sem_g0, sem_g1):
    cid = lax.axis_index("c")
    sid = lax.axis_index("s")
    wid = cid * _NS + sid
    _striped(sid, lambda lo, n: pltpu.sync_copy(
        hs_hbm.at[pl.ds(lo, n)], acc_sh.at[pl.ds(lo, n)]))
    pltpu.sync_copy(src3_hbm.at[wid], sidx_v)
    plsc.subcore_barrier()
    base = wid * _EPW

    def didx_start(j, buf, sem):
        pltpu.async_copy(dst_hbm.at[pl.ds(base + j * _CH, _CH)], buf, sem)

    def didx_wait(j, buf, sem):
        pltpu.make_async_copy(dst_hbm.at[pl.ds(base + j * _CH, _CH)], buf,
                              sem).wait()

    def g_start(j, buf, sem):
        pltpu.async_copy(hs_hbm.at[sidx_v.at[j]], buf, sem)

    def g_wait(j, buf, sem):
        pltpu.make_async_copy(hs_hbm.at[sidx_v.at[j]], buf, sem).wait()

    def scat(buf, dbuf):
        pltpu.sync_copy(buf, acc_sh.at[dbuf], add=True)

    didx_start(0, d0_v, sem_d0)
    g_start(0, rows0_v, sem_g0)
    didx_start(1, d1_v, sem_d1)

    @pl.loop(0, (_NCHUNK - 1) // 2)
    def _(k):
        g = 2 * k
        g_start(g + 1, rows1_v, sem_g1)
        g_wait(g, rows0_v, sem_g0)
        didx_wait(g, d0_v, sem_d0)
        scat(rows0_v, d0_v)

        @pl.when(g + 2 < _NCHUNK)
        def _():
            didx_start(g + 2, d0_v, sem_d0)
            g_start(g + 2, rows0_v, sem_g0)

        g_wait(g + 1, rows1_v, sem_g1)
        didx_wait(g + 1, d1_v, sem_d1)
        scat(rows1_v, d1_v)

        @pl.when(g + 3 < _NCHUNK)
        def _():
            didx_start(g + 3, d1_v, sem_d1)

    g_wait(_NCHUNK - 1, rows0_v, sem_g0)
    didx_wait(_NCHUNK - 1, d0_v, sem_d0)
    scat(rows0_v, d0_v)
    plsc.subcore_barrier()
    _striped(sid, lambda lo, n: pltpu.sync_copy(
        acc_sh.at[pl.ds(lo, n)], out_hbm.at[cid, pl.ds(lo, n)]))


def _sc_edge(hs, src3, dst):
    fn = pl.kernel(
        _edge_body,
        out_type=jax.ShapeDtypeStruct((_NC, _N, _D), jnp.float32),
        mesh=_sc_mesh(),
        scratch_types=[
            pltpu.VMEM((_NCHUNK, _CH), jnp.int32),
            pltpu.VMEM((_CH,), jnp.int32),
            pltpu.VMEM((_CH,), jnp.int32),
            pltpu.VMEM((_CH, _D), jnp.float32),
            pltpu.VMEM((_CH, _D), jnp.float32),
            pltpu.VMEM_SHARED((_N, _D), jnp.float32),
            pltpu.SemaphoreType.DMA,
            pltpu.SemaphoreType.DMA,
            pltpu.SemaphoreType.DMA,
            pltpu.SemaphoreType.DMA,
        ])
    return fn(hs, src3, dst)


def _dinv_block(degp_ref):
    deg = degp_ref[0, :, 0:1] + degp_ref[1, :, 0:1] + 1.0
    return 1.0 / jnp.sqrt(jnp.maximum(deg, 1.0))


def _k2_body(x_ref, w_ref, degp_ref, o_ref):
    dinv = _dinv_block(degp_ref)
    h = jnp.dot(x_ref[...], w_ref[...],
                preferred_element_type=jnp.float32, precision=_HIGH)
    o_ref[...] = h * dinv


def _tc_scale1(x, W1, degp):
    return pl.pallas_call(
        _k2_body,
        grid=(_NBLK,),
        in_specs=[
            pl.BlockSpec((_ROWS, _D), lambda i: (i, 0)),
            pl.BlockSpec((_D, _D), lambda i: (0, 0)),
            pl.BlockSpec((_NC, _ROWS, _D), lambda i: (0, i, 0)),
        ],
        out_specs=pl.BlockSpec((_ROWS, _D), lambda i: (i, 0)),
        out_shape=jax.ShapeDtypeStruct((_N, _D), jnp.float32),
    )(x, W1, degp)


def _k3_body(acc_ref, hs1_ref, degp_ref, w_ref, b_ref, o_ref):
    dinv = _dinv_block(degp_ref)
    agg = acc_ref[0] + acc_ref[1] - hs1_ref[...]
    out1 = jnp.maximum(agg * dinv + b_ref[...], 0.0)
    h2 = jnp.dot(out1, w_ref[...],
                 preferred_element_type=jnp.float32, precision=_HIGH)
    o_ref[...] = h2 * dinv


def _tc_mid(acc1, hs1, degp, W2, b1r):
    return pl.pallas_call(
        _k3_body,
        grid=(_NBLK,),
        in_specs=[
            pl.BlockSpec((_NC, _ROWS, _D), lambda i: (0, i, 0)),
            pl.BlockSpec((_ROWS, _D), lambda i: (i, 0)),
            pl.BlockSpec((_NC, _ROWS, _D), lambda i: (0, i, 0)),
            pl.BlockSpec((_D, _D), lambda i: (0, 0)),
            pl.BlockSpec((1, _D), lambda i: (0, 0)),
        ],
        out_specs=pl.BlockSpec((_ROWS, _D), lambda i: (i, 0)),
        out_shape=jax.ShapeDtypeStruct((_N, _D), jnp.float32),
    )(acc1, hs1, degp, W2, b1r)


def _k4_body(acc_ref, hs2_ref, degp_ref, b_ref, batch_ref,
             node_ref, gout_ref, seg_ref, cnt_ref):
    i = pl.program_id(0)
    dinv = _dinv_block(degp_ref)
    node = (acc_ref[0] + acc_ref[1] - hs2_ref[...]) * dinv + b_ref[...]
    node_ref[...] = node
    gids = lax.broadcasted_iota(jnp.int32, (_G, _ROWS), 0)
    oh = (gids == batch_ref[0]).astype(jnp.float32)
    seg_p = jnp.dot(oh, node,
                    preferred_element_type=jnp.float32, precision=_HIGH)
    cnt_p = jnp.sum(oh, axis=1, keepdims=True)

    @pl.when(i == 0)
    def _():
        seg_ref[...] = seg_p
        cnt_ref[...] = jnp.broadcast_to(cnt_p, (_G, _D))

    @pl.when(i > 0)
    def _():
        seg_ref[...] += seg_p
        cnt_ref[...] += jnp.broadcast_to(cnt_p, (_G, _D))

    @pl.when(i == _NBLK - 1)
    def _():
        gout_ref[...] = seg_ref[...] / jnp.maximum(cnt_ref[...], 1.0)


def _tc_final(acc2, hs2, degp, b2r, batch3):
    return pl.pallas_call(
        _k4_body,
        grid=(_NBLK,),
        in_specs=[
            pl.BlockSpec((_NC, _ROWS, _D), lambda i: (0, i, 0)),
            pl.BlockSpec((_ROWS, _D), lambda i: (i, 0)),
            pl.BlockSpec((_NC, _ROWS, _D), lambda i: (0, i, 0)),
            pl.BlockSpec((1, _D), lambda i: (0, 0)),
            pl.BlockSpec((1, 1, _ROWS), lambda i: (i, 0, 0)),
        ],
        out_specs=[
            pl.BlockSpec((_ROWS, _D), lambda i: (i, 0)),
            pl.BlockSpec((_G, _D), lambda i: (0, 0)),
        ],
        out_shape=[
            jax.ShapeDtypeStruct((_N, _D), jnp.float32),
            jax.ShapeDtypeStruct((_G, _D), jnp.float32),
        ],
        scratch_shapes=[
            pltpu.VMEM((_G, _D), jnp.float32),
            pltpu.VMEM((_G, _D), jnp.float32),
        ],
    )(acc2, hs2, degp, b2r, batch3)


def kernel(x, edge_index, batch, W1, b1, W2, b2):
    src3 = edge_index[0].reshape(_NW, _NCHUNK, _CH)
    dst = edge_index[1]
    dst3 = dst.reshape(_NW, _NCHUNK, _CH)
    zeros_nd = jnp.zeros((_N, _D), jnp.float32)
    ones_cd = jnp.ones((_CH, _D), jnp.float32)
    degp = _sc_degree(dst3, zeros_nd, ones_cd)
    hs1 = _tc_scale1(x, W1, degp)
    acc1 = _sc_edge(hs1, src3, dst)
    hs2 = _tc_mid(acc1, hs1, degp, W2, b1.reshape(1, _D))
    acc2 = _sc_edge(hs2, src3, dst)
    node_emb, graph_emb = _tc_final(
        acc2, hs2, degp, b2.reshape(1, _D), batch.reshape(_NBLK, 1, _ROWS))
    return node_emb, graph_emb


# speedup vs baseline: 25.6468x; 1.0033x over previous
import functools

import jax
import jax.numpy as jnp
from jax import lax
from jax.experimental import pallas as pl
from jax.experimental.pallas import tpu as pltpu
from jax.experimental.pallas import tpu_sc as plsc

_N = 10000
_E = 320000
_D = 128
_G = 16
_NC = 2
_NS = 16
_NW = _NC * _NS
_EPW = _E // _NW
_CH = 80
_NCHUNK = _EPW // _CH
_STRIPE = 624
_LAST = _N - (_NS - 1) * _STRIPE

_ROWS = 1000
_NBLK = _N // _ROWS

_HIGH = jax.lax.Precision.HIGHEST


def _striped(sid, copy):
    @pl.when(sid < _NS - 1)
    def _():
        copy(sid * _STRIPE, _STRIPE)

    @pl.when(sid == _NS - 1)
    def _():
        copy((_NS - 1) * _STRIPE, _LAST)


def _sc_mesh():
    return plsc.VectorSubcoreMesh(
        core_axis_name="c", subcore_axis_name="s",
        num_cores=_NC, num_subcores=_NS)


def _hist_body(dst3_hbm, zeros_hbm, ones_hbm, out_hbm, didx_v, ones_v, acc_sh,
               sem):
    cid = lax.axis_index("c")
    sid = lax.axis_index("s")
    wid = cid * _NS + sid
    pltpu.sync_copy(ones_hbm, ones_v)
    pltpu.sync_copy(dst3_hbm.at[wid], didx_v)
    _striped(sid, lambda lo, n: pltpu.sync_copy(
        zeros_hbm.at[pl.ds(lo, n)], acc_sh.at[pl.ds(lo, n)]))
    plsc.subcore_barrier()

    @pl.loop(0, _NCHUNK)
    def _(g):
        pltpu.async_copy(ones_v, acc_sh.at[didx_v.at[g]], sem, add=True)

    @pl.loop(0, _NCHUNK)
    def _(g):
        pltpu.make_async_copy(ones_v, acc_sh.at[didx_v.at[g]], sem).wait()

    plsc.subcore_barrier()
    _striped(sid, lambda lo, n: pltpu.sync_copy(
        acc_sh.at[pl.ds(lo, n)], out_hbm.at[cid, pl.ds(lo, n)]))


def _sc_degree(dst3, zeros_nd, ones_cd):
    fn = pl.kernel(
        _hist_body,
        out_type=jax.ShapeDtypeStruct((_NC, _N, _D), jnp.float32),
        mesh=_sc_mesh(),
        scratch_types=[
            pltpu.VMEM((_NCHUNK, _CH), jnp.int32),
            pltpu.VMEM((_CH, _D), jnp.float32),
            pltpu.VMEM_SHARED((_N, _D), jnp.float32),
            pltpu.SemaphoreType.DMA,
        ])
    return fn(dst3, zeros_nd, ones_cd)


def _edge_body(hs_hbm, src3_hbm, dst_hbm, out_hbm, sidx_v, d0_v, d1_v,
               rows0_v, rows1_v, acc_sh, sem_d0, sem_d1, sem_g0, sem_g1):
    cid = lax.axis_index("c")
    sid = lax.axis_index("s")
    wid = cid * _NS + sid
    _striped(sid, lambda lo, n: pltpu.sync_copy(
        hs_hbm.at[pl.ds(lo, n)], acc_sh.at[pl.ds(lo, n)]))
    pltpu.sync_copy(src3_hbm.at[wid], sidx_v)
    plsc.subcore_barrier()
    base = wid * _EPW

    def didx_start(j, buf, sem):
        pltpu.async_copy(dst_hbm.at[pl.ds(base + j * _CH, _CH)], buf, sem)

    def didx_wait(j, buf, sem):
        pltpu.make_async_copy(dst_hbm.at[pl.ds(base + j * _CH, _CH)], buf,
                              sem).wait()

    def g_start(j, buf, sem):
        pltpu.async_copy(hs_hbm.at[sidx_v.at[j]], buf, sem)

    def g_wait(j, buf, sem):
        pltpu.make_async_copy(hs_hbm.at[sidx_v.at[j]], buf, sem).wait()

    def scat(buf, dbuf):
        pltpu.sync_copy(buf, acc_sh.at[dbuf], add=True)

    didx_start(0, d0_v, sem_d0)
    g_start(0, rows0_v, sem_g0)
    didx_start(1, d1_v, sem_d1)

    @pl.loop(0, (_NCHUNK - 1) // 2)
    def _(k):
        g = 2 * k
        g_start(g + 1, rows1_v, sem_g1)
        g_wait(g, rows0_v, sem_g0)
        didx_wait(g, d0_v, sem_d0)
        scat(rows0_v, d0_v)

        @pl.when(g + 2 < _NCHUNK)
        def _():
            didx_start(g + 2, d0_v, sem_d0)
            g_start(g + 2, rows0_v, sem_g0)

        g_wait(g + 1, rows1_v, sem_g1)
        didx_wait(g + 1, d1_v, sem_d1)
        scat(rows1_v, d1_v)

        @pl.when(g + 3 < _NCHUNK)
        def _():
            didx_start(g + 3, d1_v, sem_d1)

    g_wait(_NCHUNK - 1, rows0_v, sem_g0)
    didx_wait(_NCHUNK - 1, d0_v, sem_d0)
    scat(rows0_v, d0_v)
    plsc.subcore_barrier()
    _striped(sid, lambda lo, n: pltpu.sync_copy(
        acc_sh.at[pl.ds(lo, n)], out_hbm.at[cid, pl.ds(lo, n)]))


def _sc_edge(hs, src3, dst):
    fn = pl.kernel(
        _edge_body,
        out_type=jax.ShapeDtypeStruct((_NC, _N, _D), jnp.float32),
        mesh=_sc_mesh(),
        scratch_types=[
            pltpu.VMEM((_NCHUNK, _CH), jnp.int32),
            pltpu.VMEM((_CH,), jnp.int32),
            pltpu.VMEM((_CH,), jnp.int32),
            pltpu.VMEM((_CH, _D), jnp.float32),
            pltpu.VMEM((_CH, _D), jnp.float32),
            pltpu.VMEM_SHARED((_N, _D), jnp.float32),
            pltpu.SemaphoreType.DMA,
            pltpu.SemaphoreType.DMA,
            pltpu.SemaphoreType.DMA,
            pltpu.SemaphoreType.DMA,
        ])
    return fn(hs, src3, dst)


def _dinv_block(degp_ref):
    deg = degp_ref[0, :, 0:1] + degp_ref[1, :, 0:1] + 1.0
    return 1.0 / jnp.sqrt(jnp.maximum(deg, 1.0))


def _k2_body(x_ref, w_ref, degp_ref, o_ref):
    dinv = _dinv_block(degp_ref)
    h = jnp.dot(x_ref[...], w_ref[...],
                preferred_element_type=jnp.float32, precision=_HIGH)
    o_ref[...] = h * dinv


def _tc_scale1(x, W1, degp):
    return pl.pallas_call(
        _k2_body,
        grid=(_NBLK,),
        in_specs=[
            pl.BlockSpec((_ROWS, _D), lambda i: (i, 0)),
            pl.BlockSpec((_D, _D), lambda i: (0, 0)),
            pl.BlockSpec((_NC, _ROWS, _D), lambda i: (0, i, 0)),
        ],
        out_specs=pl.BlockSpec((_ROWS, _D), lambda i: (i, 0)),
        out_shape=jax.ShapeDtypeStruct((_N, _D), jnp.float32),
    )(x, W1, degp)


def _k3_body(acc_ref, hs1_ref, degp_ref, w_ref, b_ref, o_ref):
    dinv = _dinv_block(degp_ref)
    agg = acc_ref[0] + acc_ref[1] - hs1_ref[...]
    out1 = jnp.maximum(agg * dinv + b_ref[...], 0.0)
    h2 = jnp.dot(out1, w_ref[...],
                 preferred_element_type=jnp.float32, precision=_HIGH)
    o_ref[...] = h2 * dinv


def _tc_mid(acc1, hs1, degp, W2, b1r):
    return pl.pallas_call(
        _k3_body,
        grid=(_NBLK,),
        in_specs=[
            pl.BlockSpec((_NC, _ROWS, _D), lambda i: (0, i, 0)),
            pl.BlockSpec((_ROWS, _D), lambda i: (i, 0)),
            pl.BlockSpec((_NC, _ROWS, _D), lambda i: (0, i, 0)),
            pl.BlockSpec((_D, _D), lambda i: (0, 0)),
            pl.BlockSpec((1, _D), lambda i: (0, 0)),
        ],
        out_specs=pl.BlockSpec((_ROWS, _D), lambda i: (i, 0)),
        out_shape=jax.ShapeDtypeStruct((_N, _D), jnp.float32),
    )(acc1, hs1, degp, W2, b1r)


def _k4_body(acc_ref, hs2_ref, degp_ref, b_ref, batch_ref,
             node_ref, gout_ref, seg_ref, cnt_ref):
    i = pl.program_id(0)
    dinv = _dinv_block(degp_ref)
    node = (acc_ref[0] + acc_ref[1] - hs2_ref[...]) * dinv + b_ref[...]
    node_ref[...] = node
    gids = lax.broadcasted_iota(jnp.int32, (_G, _ROWS), 0)
    oh = (gids == batch_ref[0]).astype(jnp.float32)
    seg_p = jnp.dot(oh, node,
                    preferred_element_type=jnp.float32, precision=_HIGH)
    cnt_p = jnp.sum(oh, axis=1, keepdims=True)

    @pl.when(i == 0)
    def _():
        seg_ref[...] = seg_p
        cnt_ref[...] = jnp.broadcast_to(cnt_p, (_G, _D))

    @pl.when(i > 0)
    def _():
        seg_ref[...] += seg_p
        cnt_ref[...] += jnp.broadcast_to(cnt_p, (_G, _D))

    @pl.when(i == _NBLK - 1)
    def _():
        gout_ref[...] = seg_ref[...] / jnp.maximum(cnt_ref[...], 1.0)


def _tc_final(acc2, hs2, degp, b2r, batch3):
    return pl.pallas_call(
        _k4_body,
        grid=(_NBLK,),
        in_specs=[
            pl.BlockSpec((_NC, _ROWS, _D), lambda i: (0, i, 0)),
            pl.BlockSpec((_ROWS, _D), lambda i: (i, 0)),
            pl.BlockSpec((_NC, _ROWS, _D), lambda i: (0, i, 0)),
            pl.BlockSpec((1, _D), lambda i: (0, 0)),
            pl.BlockSpec((1, 1, _ROWS), lambda i: (i, 0, 0)),
        ],
        out_specs=[
            pl.BlockSpec((_ROWS, _D), lambda i: (i, 0)),
            pl.BlockSpec((_G, _D), lambda i: (0, 0)),
        ],
        out_shape=[
            jax.ShapeDtypeStruct((_N, _D), jnp.float32),
            jax.ShapeDtypeStruct((_G, _D), jnp.float32),
        ],
        scratch_shapes=[
            pltpu.VMEM((_G, _D), jnp.float32),
            pltpu.VMEM((_G, _D), jnp.float32),
        ],
    )(acc2, hs2, degp, b2r, batch3)


def kernel(x, edge_index, batch, W1, b1, W2, b2):
    src3 = edge_index[0].reshape(_NW, _NCHUNK, _CH)
    dst = edge_index[1]
    dst3 = dst.reshape(_NW, _NCHUNK, _CH)
    zeros_nd = jnp.zeros((_N, _D), jnp.float32)
    ones_cd = jnp.ones((_CH, _D), jnp.float32)
    degp = _sc_degree(dst3, zeros_nd, ones_cd)
    hs1 = _tc_scale1(x, W1, degp)
    acc1 = _sc_edge(hs1, src3, dst)
    hs2 = _tc_mid(acc1, hs1, degp, W2, b1.reshape(1, _D))
    acc2 = _sc_edge(hs2, src3, dst)
    node_emb, graph_emb = _tc_final(
        acc2, hs2, degp, b2.reshape(1, _D), batch.reshape(_NBLK, 1, _ROWS))
    return node_emb, graph_emb
